# baseline (speedup 1.0000x reference)
import jax
import jax.numpy as jnp
from jax import lax
from jax.experimental import pallas as pl
from jax.experimental.pallas import tpu as pltpu
from jax.experimental.pallas import tpu_sc as plsc

B = 16384
SIZE = 64
M = 5
NEG = 5
NSLOT = 1 + NEG
CM = NSLOT * M

NC = 2
NS = 16
NW = NC * NS
L = 16

ROWS_PER_W = B // NW
C = 16
NCHUNK = ROWS_PER_W // C
WGI = C * M
CGN = 4
CGI = C * CM // CGN
NQ = SIZE // L

RWM = ROWS_PER_W * M
RCM = ROWS_PER_W * CM

TC_ROWS = 2048


def _sc_body(w2m_hbm, wmask_hbm, c2m_hbm, cmask_hbm, emb0_hbm, emb1_hbm,
             out_hbm,
             widx_all, cidx_all, wmask_all, cmask_all,
             wrows, crows, wemb_v, ips,
             sem_w, sem_c, sem_o):
    wid = lax.axis_index("s") * NC + lax.axis_index("c")
    zeros = jnp.zeros((L,), jnp.float32)

    pltpu.sync_copy(w2m_hbm.at[wid], widx_all)
    pltpu.sync_copy(c2m_hbm.at[wid], cidx_all)
    pltpu.sync_copy(wmask_hbm.at[pl.ds(wid * RWM, RWM)],
                    wmask_all.at[pl.ds(0, RWM)])
    pltpu.sync_copy(cmask_hbm.at[pl.ds(wid * RCM, RCM)],
                    cmask_all.at[pl.ds(0, RCM)])

    def issue_word(ch, b):
        pltpu.async_copy(emb0_hbm.at[widx_all.at[ch]], wrows[b], sem_w[b])

    def issue_ctx(ch, b):
        for g in range(CGN):
            pltpu.async_copy(emb1_hbm.at[cidx_all.at[ch * CGN + g]],
                             crows[b].at[pl.ds(g * CGI, CGI)], sem_c[b])

    def drain(rows_v, sem):
        pltpu.make_async_copy(emb0_hbm.at[pl.ds(0, rows_v.shape[0])],
                              rows_v, sem).wait()

    def compute_wpool(ch, rows_v):
        moff = ch * WGI

        def body(r, c2):
            i0 = r * M
            mvec = wmask_all[pl.ds(moff + i0, L)]
            acc = [zeros for _ in range(NQ)]
            for m in range(M):
                wm = mvec[m]
                for q in range(NQ):
                    acc[q] = acc[q] + wm * rows_v[i0 + m, pl.ds(q * L, L)]
            for q in range(NQ):
                wemb_v[r, pl.ds(q * L, L)] = acc[q]
            return c2

        lax.fori_loop(0, C, body, 0)

    def compute_slots(ch, rows_v, ips_v):
        moff = ch * (C * CM)

        def body(r, c2):
            i0 = r * CM
            mv0 = cmask_all[pl.ds(moff + i0, L)]
            mv1 = cmask_all[pl.ds(moff + i0 + L, L)]
            wq = [wemb_v[r, pl.ds(q * L, L)] for q in range(NQ)]
            for j in range(NSLOT):
                acc = zeros
                for m in range(M):
                    k = j * M + m
                    row0 = i0 + k
                    pm = rows_v[row0, pl.ds(0, L)] * wq[0]
                    for q in range(1, NQ):
                        pm = pm + rows_v[row0, pl.ds(q * L, L)] * wq[q]
                    cm = mv0[k] if k < L else mv1[k - L]
                    acc = acc + cm * pm
                o0 = r * (8 * L) + j * L
                ips_v[pl.ds(o0, L)] = -acc if j == 0 else acc
            ips_v[pl.ds(r * (8 * L) + 6 * L, L)] = zeros
            ips_v[pl.ds(r * (8 * L) + 7 * L, L)] = zeros
            return c2

        lax.fori_loop(0, C, body, 0)

    issue_word(0, 0)
    issue_ctx(0, 0)

    def pair_body(i, carry):
        for p in range(2):
            ch = i * 2 + p
            chn = ch + 1

            @pl.when(chn < NCHUNK)
            def _():
                issue_word(chn, 1 - p)

            drain(wrows[p], sem_w[p])
            compute_wpool(ch, wrows[p])

            @pl.when(chn < NCHUNK)
            def _():
                issue_ctx(chn, 1 - p)

            drain(crows[p], sem_c[p])

            @pl.when(ch >= 2)
            def _():
                pltpu.make_async_copy(
                    out_hbm.at[pl.ds(0, C * 8 * L)], ips[p], sem_o[p]).wait()

            compute_slots(ch, crows[p], ips[p])
            base = (wid * NCHUNK + ch) * C
            pltpu.async_copy(
                ips[p], out_hbm.at[pl.ds(base * 8 * L, C * 8 * L)], sem_o[p])
        return carry

    lax.fori_loop(0, NCHUNK // 2, pair_body, 0)

    for p in range(2):
        pltpu.make_async_copy(
            out_hbm.at[pl.ds(0, C * 8 * L)], ips[p], sem_o[p]).wait()


_sc_ips = pl.kernel(
    _sc_body,
    out_type=jax.ShapeDtypeStruct((B * 8 * L,), jnp.float32),
    mesh=plsc.VectorSubcoreMesh(core_axis_name="c", subcore_axis_name="s"),
    compiler_params=pltpu.CompilerParams(use_tc_tiling_on_sc=False),
    scratch_types=[
        pltpu.VMEM((NCHUNK, WGI), jnp.int32),
        pltpu.VMEM((NCHUNK * CGN, CGI), jnp.int32),
        pltpu.VMEM((RWM + L,), jnp.float32),
        pltpu.VMEM((RCM + 2 * L,), jnp.float32),
        [pltpu.VMEM((WGI, SIZE), jnp.float32)] * 2,
        [pltpu.VMEM((C * CM, SIZE), jnp.float32)] * 2,
        pltpu.VMEM((C, SIZE), jnp.float32),
        [pltpu.VMEM((C * 8 * L,), jnp.float32)] * 2,
        [pltpu.SemaphoreType.DMA] * 2,
        [pltpu.SemaphoreType.DMA] * 2,
        [pltpu.SemaphoreType.DMA] * 2,
    ],
)


def _loss_body(x_ref, w_ref, o_ref):
    i = lax.broadcasted_iota(jnp.int32, (128, 8), 0)
    j = lax.broadcasted_iota(jnp.int32, (128, 8), 1)
    g = jnp.where(i // L == j, 1.0, 0.0).astype(jnp.float32)
    y = jnp.dot(x_ref[...], g, preferred_element_type=jnp.float32)
    y = jnp.clip(y, -10.0, 10.0)
    part = jnp.sum(w_ref[...] * jax.nn.softplus(y))

    @pl.when(pl.program_id(0) == 0)
    def _():
        o_ref[...] = jnp.zeros_like(o_ref)

    o_ref[...] = o_ref[...] + jnp.full((1, 1), part, jnp.float32)


def _loss_tc(x2d, w2d):
    grid = (B // TC_ROWS,)
    return pl.pallas_call(
        _loss_body,
        grid=grid,
        in_specs=[
            pl.BlockSpec((TC_ROWS, 128), lambda i: (i, 0)),
            pl.BlockSpec((TC_ROWS, 8), lambda i: (i, 0)),
        ],
        out_specs=pl.BlockSpec((1, 1), lambda i: (0, 0)),
        out_shape=jax.ShapeDtypeStruct((1, 1), jnp.float32),
    )(x2d, w2d)


def kernel(data, word2morph, word2morph_mask, ctx2morph, ctx2morph_mask, emb0, emb1):
    w2m_g = word2morph.reshape(NW, NCHUNK, WGI)
    wmask = word2morph_mask.reshape(B * M)
    c2m_g = ctx2morph.reshape(NW, NCHUNK * CGN, CGI)
    cmask = ctx2morph_mask.reshape(NW * RCM)

    ips = _sc_ips(w2m_g, wmask, c2m_g, cmask, emb0, emb1)

    neg_mask = data[:, 2 + NEG:].astype(jnp.float32)
    wts = jnp.concatenate(
        [jnp.ones((B, 1), jnp.float32), neg_mask, jnp.zeros((B, 2), jnp.float32)],
        axis=1)

    loss = _loss_tc(ips.reshape(B, 8 * L), wts)
    return loss[0, 0]
